# baseline (speedup 1.0000x reference)
import functools
import jax
import jax.numpy as jnp
from jax import lax
from jax.experimental import pallas as pl
from jax.experimental.pallas import tpu as pltpu
from jax.experimental.pallas import tpu_sc as plsc

N = 100000
DIM = 128
NC = 2
NS = 16
NW = NC * NS
CH = 200
NCHUNK = N // CH
L = 16


def _sc_body(x_hbm, m_hbm, w_hbm, out_hbm, mask_v, w_v, buf_v):
    wid = lax.axis_index("s") * NC + lax.axis_index("c")

    pltpu.sync_copy(w_hbm, w_v)
    wm1 = [w_v[pl.ds(j * L, L)] - 1.0 for j in range(DIM // L)]

    def chunk_body(i, _):
        c = wid + i * NW
        row0 = c * CH
        pltpu.sync_copy(x_hbm.at[pl.ds(row0, CH)], buf_v)
        pltpu.sync_copy(m_hbm.at[pl.ds(row0, CH)], mask_v)

        def row_body(r, _):
            mvec = plsc.load_gather(mask_v, [jnp.full((L,), r, jnp.int32)])
            for j in range(DIM // L):
                xv = buf_v[r, pl.ds(j * L, L)]
                buf_v[r, pl.ds(j * L, L)] = xv * (mvec * wm1[j] + 1.0)
            return 0

        lax.fori_loop(0, CH, row_body, 0)
        pltpu.sync_copy(buf_v, out_hbm.at[pl.ds(row0, CH)])
        return 0

    n_mine = (NCHUNK - 1 - wid) // NW + 1
    lax.fori_loop(0, n_mine, chunk_body, 0)


def kernel(x, node_mask, deletion_weight):
    m = node_mask.astype(jnp.float32)
    mesh = plsc.VectorSubcoreMesh(core_axis_name="c", subcore_axis_name="s")
    k = functools.partial(
        pl.kernel,
        out_type=jax.ShapeDtypeStruct((N, DIM), jnp.float32),
        mesh=mesh,
        compiler_params=pltpu.CompilerParams(needs_layout_passes=False),
        scratch_types=[
            pltpu.VMEM((CH,), jnp.float32),
            pltpu.VMEM((DIM,), jnp.float32),
            pltpu.VMEM((CH, DIM), jnp.float32),
        ],
    )(_sc_body)
    return k(x, m, deletion_weight)


# speedup vs baseline: 1.3912x; 1.3912x over previous
import functools
import jax
import jax.numpy as jnp
from jax import lax
from jax.experimental import pallas as pl
from jax.experimental.pallas import tpu as pltpu
from jax.experimental.pallas import tpu_sc as plsc

N = 100000
DIM = 128
NC = 2
NS = 16
NW = NC * NS
CH = 400
NCHUNK = N // CH
NK = -(-NCHUNK // NW)
L = 16


def _sc_body(x_hbm, m_hbm, w_hbm, out_hbm,
             b0, b1, mb0, mb1, w_v,
             ls0, ls1, ms0, ms1, ss0, ss1):
    wid = lax.axis_index("s") * NC + lax.axis_index("c")

    pltpu.sync_copy(w_hbm, w_v)
    wm1 = [w_v[pl.ds(j * L, L)] - 1.0 for j in range(DIM // L)]

    bufs = (b0, b1)
    mbufs = (mb0, mb1)
    lsems = (ls0, ls1)
    msems = (ms0, ms1)
    ssems = (ss0, ss1)

    def row0_of(k):
        c = jnp.minimum(wid + k * NW, NCHUNK - 1)
        return c * CH

    def issue_load(k):
        b = k % 2
        r0 = row0_of(k)
        hx = pltpu.async_copy(x_hbm.at[pl.ds(r0, CH)], bufs[b], lsems[b])
        hm = pltpu.async_copy(m_hbm.at[pl.ds(r0, CH)], mbufs[b], msems[b])
        return hx, hm

    def compute(k):
        b = k % 2
        buf, mbuf = bufs[b], mbufs[b]

        def row_body(r, _):
            mvec = plsc.load_gather(mbuf, [jnp.full((L,), r, jnp.int32)])
            for j in range(DIM // L):
                xv = buf[r, pl.ds(j * L, L)]
                buf[r, pl.ds(j * L, L)] = xv * (mvec * wm1[j] + 1.0)
            return 0

        lax.fori_loop(0, CH, row_body, 0)

    pending_loads = {0: issue_load(0)}
    pending_stores = {}
    for k in range(NK):
        b = k % 2
        if k >= 1:
            pending_stores.pop(k - 1).wait()
        if k + 1 < NK:
            pending_loads[k + 1] = issue_load(k + 1)
        hx, hm = pending_loads.pop(k)
        hx.wait()
        hm.wait()
        compute(k)
        pending_stores[k] = pltpu.async_copy(
            bufs[b], out_hbm.at[pl.ds(row0_of(k), CH)], ssems[b])
    pending_stores.pop(NK - 1).wait()


def kernel(x, node_mask, deletion_weight):
    m = node_mask.astype(jnp.float32)
    mesh = plsc.VectorSubcoreMesh(core_axis_name="c", subcore_axis_name="s")
    k = functools.partial(
        pl.kernel,
        out_type=jax.ShapeDtypeStruct((N, DIM), jnp.float32),
        mesh=mesh,
        compiler_params=pltpu.CompilerParams(needs_layout_passes=False),
        scratch_types=[
            pltpu.VMEM((CH, DIM), jnp.float32),
            pltpu.VMEM((CH, DIM), jnp.float32),
            pltpu.VMEM((CH,), jnp.float32),
            pltpu.VMEM((CH,), jnp.float32),
            pltpu.VMEM((DIM,), jnp.float32),
            pltpu.SemaphoreType.DMA,
            pltpu.SemaphoreType.DMA,
            pltpu.SemaphoreType.DMA,
            pltpu.SemaphoreType.DMA,
            pltpu.SemaphoreType.DMA,
            pltpu.SemaphoreType.DMA,
        ],
    )(_sc_body)
    return k(x, m, deletion_weight)


# speedup vs baseline: 1.4766x; 1.0614x over previous
import functools
import jax
import jax.numpy as jnp
from jax import lax
from jax.experimental import pallas as pl
from jax.experimental.pallas import tpu as pltpu
from jax.experimental.pallas import tpu_sc as plsc

N = 100000
DIM = 128
NC = 2
NS = 16
NW = NC * NS
CH = 400
NCHUNK = N // CH
NK = -(-NCHUNK // NW)
L = 16


def _sc_body(x_hbm, m_hbm, w_hbm, out_hbm,
             b0, b1, mb0, mb1, w_v,
             ls0, ls1, ms0, ms1, ss0, ss1):
    wid = lax.axis_index("s") * NC + lax.axis_index("c")

    pltpu.sync_copy(w_hbm, w_v)
    wv = [w_v[pl.ds(j * L, L)] for j in range(DIM // L)]

    bufs = (b0, b1)
    mbufs = (mb0, mb1)
    lsems = (ls0, ls1)
    msems = (ms0, ms1)
    ssems = (ss0, ss1)

    def row0_of(k):
        c = jnp.minimum(wid + k * NW, NCHUNK - 1)
        return c * CH

    def issue_load(k):
        b = k % 2
        r0 = row0_of(k)
        hx = pltpu.async_copy(x_hbm.at[pl.ds(r0, CH)], bufs[b], lsems[b])
        hm = pltpu.async_copy(m_hbm.at[pl.ds(r0, CH)], mbufs[b], msems[b])
        return hx, hm

    def compute(k):
        b = k % 2
        buf, mbuf = bufs[b], mbufs[b]

        UNROLL = 4

        def row_body(r4, _):
            for u in range(UNROLL):
                r = r4 * UNROLL + u
                mvec = plsc.load_gather(mbuf, [jnp.full((L,), r, jnp.int32)])
                keep = mvec > 0.0
                for j in range(DIM // L):
                    xv = buf[r, pl.ds(j * L, L)]
                    buf[r, pl.ds(j * L, L)] = jnp.where(keep, xv * wv[j], xv)
            return 0

        lax.fori_loop(0, CH // UNROLL, row_body, 0)

    pending_loads = {0: issue_load(0)}
    pending_stores = {}
    for k in range(NK):
        b = k % 2
        if k >= 1:
            pending_stores.pop(k - 1).wait()
        if k + 1 < NK:
            pending_loads[k + 1] = issue_load(k + 1)
        hx, hm = pending_loads.pop(k)
        hx.wait()
        hm.wait()
        compute(k)
        pending_stores[k] = pltpu.async_copy(
            bufs[b], out_hbm.at[pl.ds(row0_of(k), CH)], ssems[b])
    pending_stores.pop(NK - 1).wait()


def kernel(x, node_mask, deletion_weight):
    m = node_mask.astype(jnp.float32)
    mesh = plsc.VectorSubcoreMesh(core_axis_name="c", subcore_axis_name="s")
    k = functools.partial(
        pl.kernel,
        out_type=jax.ShapeDtypeStruct((N, DIM), jnp.float32),
        mesh=mesh,
        compiler_params=pltpu.CompilerParams(needs_layout_passes=False),
        scratch_types=[
            pltpu.VMEM((CH, DIM), jnp.float32),
            pltpu.VMEM((CH, DIM), jnp.float32),
            pltpu.VMEM((CH,), jnp.float32),
            pltpu.VMEM((CH,), jnp.float32),
            pltpu.VMEM((DIM,), jnp.float32),
            pltpu.SemaphoreType.DMA,
            pltpu.SemaphoreType.DMA,
            pltpu.SemaphoreType.DMA,
            pltpu.SemaphoreType.DMA,
            pltpu.SemaphoreType.DMA,
            pltpu.SemaphoreType.DMA,
        ],
    )(_sc_body)
    return k(x, m, deletion_weight)
